# Initial kernel scaffold; baseline (speedup 1.0000x reference)
import jax
import jax.numpy as jnp
from jax.experimental import pallas as pl


def kernel(node_feature, relative_pos):
    raise NotImplementedError("write your pallas kernel here")


# speedup vs baseline: 7.7337x; 7.7337x over previous
import functools

import jax
import jax.numpy as jnp
from jax import lax
from jax.experimental import pallas as pl
from jax.experimental.pallas import tpu as pltpu

INF = 100000.0
DIM = 96
RES = 32
NUM_PATCH = RES * RES
K = 10
BATCH = 32
RB = 256
NB = NUM_PATCH // RB


def _body(feat_ref, rel_ref, out_ref):
    r = pl.program_id(0)
    b = pl.program_id(1)
    x = feat_ref[0]
    nrm = jnp.sqrt(jnp.sum(x * x, axis=1, keepdims=True))
    xn = x / jnp.clip(nrm, 1e-12, None)
    s = jnp.sum(xn * xn, axis=1)
    xr_raw = feat_ref[0, pl.ds(r * RB, RB), :]
    nrm_r = jnp.sqrt(jnp.sum(xr_raw * xr_raw, axis=1, keepdims=True))
    xr = xr_raw / jnp.clip(nrm_r, 1e-12, None)
    sr = jnp.sum(xr * xr, axis=1)
    prod = lax.dot_general(xr, xn, (((1,), (1,)), ((), ())),
                           preferred_element_type=jnp.float32)
    dist = sr[:, None] + s[None, :] - 2.0 * prod + rel_ref[0]
    gi = r * RB + lax.broadcasted_iota(jnp.int32, (RB, NUM_PATCH), 0)
    gj = lax.broadcasted_iota(jnp.int32, (RB, NUM_PATCH), 1)
    nbr = (jnp.abs((gi >> 5) - (gj >> 5)) <= 1) & (jnp.abs((gi & 31) - (gj & 31)) <= 1)
    dist = jnp.where(nbr, dist + INF, dist)

    cols = gj
    outs = []
    for _ in range(K):
        m = jnp.min(dist, axis=1, keepdims=True)
        idx = jnp.min(jnp.where(dist == m, cols, jnp.int32(2**30)), axis=1)
        outs.append(idx + b * NUM_PATCH)
        dist = jnp.where(cols == idx[:, None], jnp.float32(jnp.inf), dist)
    out_ref[0] = jnp.stack(outs, axis=1)


@functools.partial(jax.jit, static_argnums=())
def _topk_call(node_feature, relative_pos):
    return pl.pallas_call(
        _body,
        grid=(NB, BATCH),
        in_specs=[
            pl.BlockSpec((1, NUM_PATCH, DIM), lambda r, b: (b, 0, 0)),
            pl.BlockSpec((1, RB, NUM_PATCH), lambda r, b: (0, r, 0)),
        ],
        out_specs=pl.BlockSpec((1, RB, K), lambda r, b: (b, r, 0)),
        out_shape=jax.ShapeDtypeStruct((BATCH, NUM_PATCH, K), jnp.int32),
    )(node_feature, relative_pos)


def kernel(node_feature, relative_pos):
    b, n, _ = node_feature.shape
    tk = _topk_call(node_feature, relative_pos)
    src = jnp.broadcast_to(
        jnp.arange(b * n, dtype=jnp.int32).reshape(b, n, 1), (b, n, K))
    edge_list = jnp.stack([tk, src], axis=-1).reshape(-1, 2)
    relation = jnp.zeros((edge_list.shape[0], 1), dtype=edge_list.dtype)
    edge_list = jnp.concatenate([edge_list, relation], axis=-1)
    return (edge_list, 1)
